# baseline (speedup 1.0000x reference)
import jax
import jax.numpy as jnp
from jax import lax
from jax.experimental import pallas as pl
from jax.experimental.pallas import tpu as pltpu
from jax.experimental.pallas import tpu_sc as plsc

B = 4096
L = 200
D = 64
NW = 32
BC = B // NW
LANES = 16
HALF_L = L // 2


def _body(xT_hbm, tok_hbm, pos_hbm, outT_hbm, x_v, j_v, pos_v, g_v, t_v,
          sg0, sg1, sw0, sw1):
    wid = lax.axis_index("s") * 2 + lax.axis_index("c")
    semg = (sg0, sg1)
    semw = (sw0, sw1)
    pltpu.sync_copy(xT_hbm.at[:, pl.ds(wid * BC, BC)], x_v)
    pltpu.sync_copy(pos_hbm, pos_v)

    def jrow(l, carry):
        for cc in range(BC // LANES):
            xv = x_v[l, pl.ds(cc * LANES, LANES)]
            j_v[l, pl.ds(cc * LANES, LANES)] = lax.shift_right_logical(xv, 1)
        return carry

    lax.fori_loop(0, L, jrow, 0)

    def fire(l, p):
        pltpu.async_copy(tok_hbm.at[j_v.at[l]], g_v.at[p], semg[p])

    def drain_g(p):
        pltpu.make_async_copy(tok_hbm.at[pl.ds(0, BC)], g_v.at[p],
                              semg[p]).wait()

    def drain_w(p):
        pltpu.make_async_copy(t_v.at[p], outT_hbm.at[0, :, pl.ds(0, BC)],
                              semw[p]).wait()

    iota = lax.iota(jnp.int32, LANES)

    def process(l, p):
        hcols = [(x_v[l, pl.ds(cc * LANES, LANES)] & 1) * D
                 for cc in range(BC // LANES)]

        def dbody(d, carry):
            pv = plsc.load_gather(
                pos_v, [jnp.zeros((LANES,), jnp.int32) + (l * D + d)])
            for cc in range(BC // LANES):
                gv = plsc.load_gather(
                    g_v.at[p], [iota + cc * LANES, hcols[cc] + d])
                t_v[p, d, pl.ds(cc * LANES, LANES)] = gv + pv
            return carry

        lax.fori_loop(0, D, dbody, 0)
        pltpu.async_copy(t_v.at[p], outT_hbm.at[l, :, pl.ds(wid * BC, BC)],
                         semw[p])

    fire(0, 0)

    def step(t, carry):
        @pl.when(t > 0)
        def _():
            drain_w(1)

        fire(2 * t + 1, 1)
        drain_g(0)
        process(2 * t, 0)

        @pl.when(t < HALF_L - 1)
        def _():
            drain_w(0)
            fire(2 * t + 2, 0)

        drain_g(1)
        process(2 * t + 1, 1)
        return carry

    lax.fori_loop(0, HALF_L, step, 0)
    drain_w(0)
    drain_w(1)


@jax.jit
def _embed(xT, tok128, pos_flat):
    mesh = plsc.VectorSubcoreMesh(core_axis_name="c", subcore_axis_name="s")
    kfn = pl.kernel(
        _body,
        out_type=jax.ShapeDtypeStruct((L, D, B), jnp.float32),
        mesh=mesh,
        scratch_types=[
            pltpu.VMEM((L, BC), jnp.int32),
            pltpu.VMEM((L, BC), jnp.int32),
            pltpu.VMEM((L * D,), jnp.float32),
            pltpu.VMEM((2, BC, 2 * D), jnp.float32),
            pltpu.VMEM((2, D, BC), jnp.float32),
            pltpu.SemaphoreType.DMA,
            pltpu.SemaphoreType.DMA,
            pltpu.SemaphoreType.DMA,
            pltpu.SemaphoreType.DMA,
        ],
        compiler_params=pltpu.CompilerParams(use_tc_tiling_on_sc=True,
                                             needs_layout_passes=False),
    )
    return kfn(xT, tok128, pos_flat)


def kernel(x, token_table, pos_table):
    xT = x.T.astype(jnp.int32)
    tok128 = token_table.reshape(500000, 128)
    pos_flat = pos_table.reshape(L * D)
    outT = _embed(xT, tok128, pos_flat)
    return jnp.transpose(outT, (2, 0, 1))


# speedup vs baseline: 1.5014x; 1.5014x over previous
import jax
import jax.numpy as jnp
from jax import lax
from jax.experimental import pallas as pl
from jax.experimental.pallas import tpu as pltpu
from jax.experimental.pallas import tpu_sc as plsc

B = 4096
L = 200
D = 64
NW = 32
BC = B // NW
LANES = 16
HALF_L = L // 2


def _body(xT_hbm, tok_hbm, pos_hbm, outT_hbm, x_v, j_v, pos_v, g_v, t_v,
          sg0, sg1, sw0, sw1):
    wid = lax.axis_index("s") * 2 + lax.axis_index("c")
    semg = (sg0, sg1)
    semw = (sw0, sw1)
    pltpu.sync_copy(xT_hbm.at[:, pl.ds(wid * BC, BC)], x_v)
    pltpu.sync_copy(pos_hbm, pos_v)

    def jrow(l, carry):
        for cc in range(BC // LANES):
            xv = x_v[l, pl.ds(cc * LANES, LANES)]
            j_v[l, pl.ds(cc * LANES, LANES)] = lax.shift_right_logical(xv, 1)
        return carry

    lax.fori_loop(0, L, jrow, 0)

    def fire(l, p):
        pltpu.async_copy(tok_hbm.at[j_v.at[l]], g_v.at[p], semg[p])

    def drain_g(p):
        pltpu.make_async_copy(tok_hbm.at[pl.ds(0, BC)], g_v.at[p],
                              semg[p]).wait()

    def drain_w(p):
        pltpu.make_async_copy(t_v.at[p], outT_hbm.at[0, :, pl.ds(0, BC)],
                              semw[p]).wait()

    iota = lax.iota(jnp.int32, LANES)

    def process(l, p):
        hcols = [(x_v[l, pl.ds(cc * LANES, LANES)] & 1) * D
                 for cc in range(BC // LANES)]

        @plsc.parallel_loop(0, D, unroll=4)
        def _(d):
            pv = plsc.load_gather(
                pos_v, [jnp.zeros((LANES,), jnp.int32) + (l * D + d)])
            for cc in range(BC // LANES):
                gv = plsc.load_gather(
                    g_v.at[p], [iota + cc * LANES, hcols[cc] + d])
                t_v[p, d, pl.ds(cc * LANES, LANES)] = gv + pv
        pltpu.async_copy(t_v.at[p], outT_hbm.at[l, :, pl.ds(wid * BC, BC)],
                         semw[p])

    fire(0, 0)

    def step(t, carry):
        @pl.when(t > 0)
        def _():
            drain_w(1)

        fire(2 * t + 1, 1)
        drain_g(0)
        process(2 * t, 0)

        @pl.when(t < HALF_L - 1)
        def _():
            drain_w(0)
            fire(2 * t + 2, 0)

        drain_g(1)
        process(2 * t + 1, 1)
        return carry

    lax.fori_loop(0, HALF_L, step, 0)
    drain_w(0)
    drain_w(1)


@jax.jit
def _embed(xT, tok128, pos_flat):
    mesh = plsc.VectorSubcoreMesh(core_axis_name="c", subcore_axis_name="s")
    kfn = pl.kernel(
        _body,
        out_type=jax.ShapeDtypeStruct((L, D, B), jnp.float32),
        mesh=mesh,
        scratch_types=[
            pltpu.VMEM((L, BC), jnp.int32),
            pltpu.VMEM((L, BC), jnp.int32),
            pltpu.VMEM((L * D,), jnp.float32),
            pltpu.VMEM((2, BC, 2 * D), jnp.float32),
            pltpu.VMEM((2, D, BC), jnp.float32),
            pltpu.SemaphoreType.DMA,
            pltpu.SemaphoreType.DMA,
            pltpu.SemaphoreType.DMA,
            pltpu.SemaphoreType.DMA,
        ],
        compiler_params=pltpu.CompilerParams(use_tc_tiling_on_sc=True,
                                             needs_layout_passes=False),
    )
    return kfn(xT, tok128, pos_flat)


def kernel(x, token_table, pos_table):
    xT = x.T.astype(jnp.int32)
    tok128 = token_table.reshape(500000, 128)
    pos_flat = pos_table.reshape(L * D)
    outT = _embed(xT, tok128, pos_flat)
    return jnp.transpose(outT, (2, 0, 1))


# speedup vs baseline: 1.5062x; 1.0032x over previous
import jax
import jax.numpy as jnp
from jax import lax
from jax.experimental import pallas as pl
from jax.experimental.pallas import tpu as pltpu
from jax.experimental.pallas import tpu_sc as plsc

B = 4096
L = 200
D = 64
NW = 32
BC = B // NW
LANES = 16
HALF_L = L // 2


def _body(xT_hbm, tok_hbm, pos_hbm, outT_hbm, x_v, j_v, pos_v, g_v, t_v,
          sg0, sg1, sw0, sw1):
    wid = lax.axis_index("s") * 2 + lax.axis_index("c")
    semg = (sg0, sg1)
    semw = (sw0, sw1)
    pltpu.sync_copy(xT_hbm.at[:, pl.ds(wid * BC, BC)], x_v)
    pltpu.sync_copy(pos_hbm, pos_v)

    def jrow(l, carry):
        for cc in range(BC // LANES):
            xv = x_v[l, pl.ds(cc * LANES, LANES)]
            j_v[l, pl.ds(cc * LANES, LANES)] = lax.shift_right_logical(xv, 1)
        return carry

    lax.fori_loop(0, L, jrow, 0)

    def fire(l, p):
        pltpu.async_copy(tok_hbm.at[j_v.at[l]], g_v.at[p], semg[p])

    def drain_g(p):
        pltpu.make_async_copy(tok_hbm.at[pl.ds(0, BC)], g_v.at[p],
                              semg[p]).wait()

    def drain_w(p):
        pltpu.make_async_copy(t_v.at[p], outT_hbm.at[0, :, pl.ds(0, BC)],
                              semw[p]).wait()

    iota = lax.iota(jnp.int32, LANES)

    zeros = jnp.zeros((LANES,), jnp.int32)
    rows16 = [iota + cc * LANES for cc in range(BC // LANES)]

    def process(l, p):
        hcols = [(x_v[l, pl.ds(cc * LANES, LANES)] & 1) * D
                 for cc in range(BC // LANES)]
        lD = l * D

        @plsc.parallel_loop(0, D, unroll=8)
        def _(d):
            pv = plsc.load_gather(pos_v, [zeros + (lD + d)])
            for cc in range(BC // LANES):
                gv = plsc.load_gather(
                    g_v.at[p], [rows16[cc], hcols[cc] + d])
                t_v[p, d, pl.ds(cc * LANES, LANES)] = gv + pv
        pltpu.async_copy(t_v.at[p], outT_hbm.at[l, :, pl.ds(wid * BC, BC)],
                         semw[p])

    fire(0, 0)

    def step(t, carry):
        @pl.when(t > 0)
        def _():
            drain_w(1)

        fire(2 * t + 1, 1)
        drain_g(0)
        process(2 * t, 0)

        @pl.when(t < HALF_L - 1)
        def _():
            drain_w(0)
            fire(2 * t + 2, 0)

        drain_g(1)
        process(2 * t + 1, 1)
        return carry

    lax.fori_loop(0, HALF_L, step, 0)
    drain_w(0)
    drain_w(1)


@jax.jit
def _embed(xT, tok128, pos_flat):
    mesh = plsc.VectorSubcoreMesh(core_axis_name="c", subcore_axis_name="s")
    kfn = pl.kernel(
        _body,
        out_type=jax.ShapeDtypeStruct((L, D, B), jnp.float32),
        mesh=mesh,
        scratch_types=[
            pltpu.VMEM((L, BC), jnp.int32),
            pltpu.VMEM((L, BC), jnp.int32),
            pltpu.VMEM((L * D,), jnp.float32),
            pltpu.VMEM((2, BC, 2 * D), jnp.float32),
            pltpu.VMEM((2, D, BC), jnp.float32),
            pltpu.SemaphoreType.DMA,
            pltpu.SemaphoreType.DMA,
            pltpu.SemaphoreType.DMA,
            pltpu.SemaphoreType.DMA,
        ],
        compiler_params=pltpu.CompilerParams(use_tc_tiling_on_sc=True,
                                             needs_layout_passes=False),
    )
    return kfn(xT, tok128, pos_flat)


def kernel(x, token_table, pos_table):
    xT = x.T.astype(jnp.int32)
    tok128 = token_table.reshape(500000, 128)
    pos_flat = pos_table.reshape(L * D)
    outT = _embed(xT, tok128, pos_flat)
    return jnp.transpose(outT, (2, 0, 1))
